# Initial kernel scaffold; baseline (speedup 1.0000x reference)
import jax
import jax.numpy as jnp
from jax.experimental import pallas as pl


def kernel(x, edge_index, src, dst, Ws0, Wn0, b0, Ws1, Wn1, b1):
    raise NotImplementedError("write your pallas kernel here")


# speedup vs baseline: 7.0669x; 7.0669x over previous
import functools

import jax
import jax.numpy as jnp
from jax import lax
from jax.experimental import pallas as pl
from jax.experimental.pallas import tpu as pltpu
from jax.experimental.pallas import tpu_sc as plsc

N_NODES = 10000
D = 128
N_EDGES = 320000
NP = 10240
NC = 2
NS = 16
NT = NC * NS
EPT = N_EDGES // NT
BE = 80
NB = EPT // BE
NCH = 5
NBC = NB // NCH
RPT = NP // NS

_mesh = plsc.VectorSubcoreMesh(core_axis_name="c", subcore_axis_name="s",
                               num_cores=NC, num_subcores=NS)


def _make_agg(with_deg: bool):
    if with_deg:
        out_type = [jax.ShapeDtypeStruct((NC, NP, D), jnp.float32),
                    jax.ShapeDtypeStruct((NC, NP), jnp.float32)]
    else:
        out_type = jax.ShapeDtypeStruct((NC, NP, D), jnp.float32)
    scratch = [
        pltpu.VMEM((NBC, BE), jnp.int32),
        pltpu.VMEM((NBC, BE), jnp.int32),
        pltpu.VMEM((BE, D), jnp.float32),
        pltpu.VMEM((BE,), jnp.float32),
        pltpu.VMEM((64, D), jnp.float32),
        pltpu.VMEM((RPT,), jnp.float32),
        pltpu.VMEM_SHARED((NP, D), jnp.float32),
        pltpu.VMEM_SHARED((NP,), jnp.float32),
        pltpu.SemaphoreType.DMA,
    ]

    def body(h_hbm, esrc_hbm, edst_hbm, *refs):
        if with_deg:
            out_hbm, deg_hbm = refs[0], refs[1]
            rest = refs[2:]
        else:
            out_hbm = refs[0]
            rest = refs[1:]
        src_idx, dst_idx, rows, ones_v, zbuf, zdeg, acc_sh, deg_sh, sem = rest
        cid = lax.axis_index("c")
        sid = lax.axis_index("s")
        wid = sid * NC + cid

        z16 = jnp.zeros((16,), jnp.float32)

        def fill_zbuf(i, _):
            for j in range(D // 16):
                zbuf[i, pl.ds(j * 16, 16)] = z16
            return 0
        lax.fori_loop(0, 64, fill_zbuf, 0)

        def fill_zdeg(i, _):
            zdeg[pl.ds(i * 16, 16)] = z16
            return 0
        lax.fori_loop(0, RPT // 16, fill_zdeg, 0)

        if with_deg:
            one16 = jnp.ones((16,), jnp.float32)
            for j in range(BE // 16):
                ones_v[pl.ds(j * 16, 16)] = one16

        for k in range(RPT // 64):
            pltpu.sync_copy(zbuf, acc_sh.at[pl.ds(sid * RPT + k * 64, 64)])
        pltpu.sync_copy(zdeg, deg_sh.at[pl.ds(sid * RPT, RPT)])

        plsc.subcore_barrier()

        def chunk(ch, _):
            pltpu.sync_copy(esrc_hbm.at[wid, ch], src_idx)
            pltpu.sync_copy(edst_hbm.at[wid, ch], dst_idx)

            def batch(b, _):
                pltpu.async_copy(h_hbm.at[src_idx.at[b]], rows, sem).wait()
                pltpu.sync_copy(rows, acc_sh.at[dst_idx.at[b]], add=True)
                if with_deg:
                    pltpu.sync_copy(ones_v, deg_sh.at[dst_idx.at[b]], add=True)
                return 0
            lax.fori_loop(0, NBC, batch, 0)
            return 0
        lax.fori_loop(0, NCH, chunk, 0)

        plsc.subcore_barrier()

        for k in range(RPT // 64):
            sl = pl.ds(sid * RPT + k * 64, 64)
            pltpu.sync_copy(acc_sh.at[sl], zbuf)
            pltpu.sync_copy(zbuf, out_hbm.at[cid, sl])
        if with_deg:
            dsl = pl.ds(sid * RPT, RPT)
            pltpu.sync_copy(deg_sh.at[dsl], zdeg)
            pltpu.sync_copy(zdeg, deg_hbm.at[cid, dsl])

    return pl.kernel(body, out_type=out_type, mesh=_mesh,
                     scratch_types=scratch)


_agg_deg = _make_agg(with_deg=True)
_agg = _make_agg(with_deg=False)


def _make_dense(do_relu: bool):
    BLK = 1024

    def body(x_ref, p_ref, d_ref, ws_ref, wn_ref, b_ref, o_ref):
        deg = d_ref[0, :] + d_ref[1, :]
        recip = 1.0 / jnp.maximum(deg, 1.0)
        neigh = (p_ref[0] + p_ref[1]) * recip[:, None]
        h = (jnp.dot(x_ref[...], ws_ref[...], preferred_element_type=jnp.float32)
             + jnp.dot(neigh, wn_ref[...], preferred_element_type=jnp.float32)
             + b_ref[...][None, :])
        if do_relu:
            h = jnp.maximum(h, 0.0)
        o_ref[...] = h

    return pl.pallas_call(
        body,
        grid=(NP // BLK,),
        in_specs=[
            pl.BlockSpec((BLK, D), lambda i: (i, 0)),
            pl.BlockSpec((NC, BLK, D), lambda i: (0, i, 0)),
            pl.BlockSpec((NC, BLK), lambda i: (0, i)),
            pl.BlockSpec((D, D), lambda i: (0, 0)),
            pl.BlockSpec((D, D), lambda i: (0, 0)),
            pl.BlockSpec((D,), lambda i: (0,)),
        ],
        out_specs=pl.BlockSpec((BLK, D), lambda i: (i, 0)),
        out_shape=jax.ShapeDtypeStruct((NP, D), jnp.float32),
    )


_dense_relu = _make_dense(True)
_dense_lin = _make_dense(False)


def _make_score():
    scratch = [
        pltpu.VMEM((NB, BE), jnp.int32),
        pltpu.VMEM((NB, BE), jnp.int32),
        pltpu.VMEM((BE, D), jnp.float32),
        pltpu.VMEM((BE, D), jnp.float32),
        pltpu.VMEM((EPT,), jnp.float32),
        pltpu.SemaphoreType.DMA,
        pltpu.SemaphoreType.DMA,
    ]

    def body(h_hbm, src_hbm, dst_hbm, out_hbm,
             src_idx, dst_idx, sbuf, dbuf, obuf, sem_s, sem_d):
        cid = lax.axis_index("c")
        sid = lax.axis_index("s")
        wid = sid * NC + cid

        pltpu.sync_copy(src_hbm.at[wid], src_idx)
        pltpu.sync_copy(dst_hbm.at[wid], dst_idx)

        lane = lax.iota(jnp.int32, 16)

        def batch(b, _):
            cs = pltpu.async_copy(h_hbm.at[src_idx.at[b]], sbuf, sem_s)
            cd = pltpu.async_copy(h_hbm.at[dst_idx.at[b]], dbuf, sem_d)
            cs.wait()
            cd.wait()

            def grp(g, _):
                def row16(k, accv):
                    r = g * 16 + k
                    acc = sbuf[r, pl.ds(0, 16)] * dbuf[r, pl.ds(0, 16)]
                    for c in range(1, D // 16):
                        acc = acc + sbuf[r, pl.ds(c * 16, 16)] * dbuf[r, pl.ds(c * 16, 16)]
                    return jnp.where(lane == k, jnp.sum(acc), accv)
                accv = lax.fori_loop(0, 16, row16, jnp.zeros((16,), jnp.float32))
                obuf[pl.ds(b * BE + g * 16, 16)] = accv
                return 0
            lax.fori_loop(0, BE // 16, grp, 0)
            return 0
        lax.fori_loop(0, NB, batch, 0)

        def sig(g, _):
            v = obuf[pl.ds(g * 16, 16)]
            obuf[pl.ds(g * 16, 16)] = 1.0 / (1.0 + jnp.exp(-v))
            return 0
        lax.fori_loop(0, EPT // 16, sig, 0)

        pltpu.sync_copy(obuf, out_hbm.at[pl.ds(wid * EPT, EPT)])

    return pl.kernel(body,
                     out_type=jax.ShapeDtypeStruct((N_EDGES,), jnp.float32),
                     mesh=_mesh, scratch_types=scratch,
                     compiler_params=pltpu.CompilerParams(
                         needs_layout_passes=False))


_score = _make_score()


def kernel(x, edge_index, src, dst, Ws0, Wn0, b0, Ws1, Wn1, b1):
    x_pad = jnp.pad(x, ((0, NP - N_NODES), (0, 0)))
    esrc = edge_index[0].reshape(NT, NB, BE)
    edst = edge_index[1].reshape(NT, NB, BE)

    src2 = src.reshape(NT, NB, BE)
    dst2 = dst.reshape(NT, NB, BE)

    esrc4 = esrc.reshape(NT, NCH, NBC, BE)
    edst4 = edst.reshape(NT, NCH, NBC, BE)
    part0, degp = _agg_deg(x_pad, esrc4, edst4)
    h0 = _dense_relu(x_pad, part0, degp, Ws0, Wn0, b0)
    part1 = _agg(h0, esrc4, edst4)
    h1 = _dense_lin(h0, part1, degp, Ws1, Wn1, b1)
    return _score(h1, src2, dst2)
